# Initial kernel scaffold; baseline (speedup 1.0000x reference)
import jax
import jax.numpy as jnp
from jax.experimental import pallas as pl


def kernel(z, constellation):
    raise NotImplementedError("write your pallas kernel here")


# speedup vs baseline: 5.4631x; 5.4631x over previous
import jax
import jax.numpy as jnp
from jax import lax
from jax.experimental import pallas as pl
from jax.experimental.pallas import tpu as pltpu
from jax.experimental.pallas import tpu_sc as plsc

_NC, _NS = 2, 16
_NW = _NC * _NS
_B, _T = 16384, 200
_N = _B * _T
_PER_W = _N // _NW
_CH = 12800
_NCH = _PER_W // _CH


def _psk_body(z_hbm, tab_hbm, out_hbm, tab_v, z_v, out_v):
    wid = lax.axis_index("s") * _NC + lax.axis_index("c")
    base = wid * _PER_W
    pltpu.sync_copy(tab_hbm, tab_v)
    iota2 = lax.iota(jnp.int32, 16) * 2

    @pl.loop(0, _NCH)
    def _chunk(ci):
        off = base + ci * _CH
        pltpu.sync_copy(z_hbm.at[pl.ds(off, _CH)], z_v)

        @pl.loop(0, _CH // 16)
        def _grp(g):
            zv = z_v[pl.ds(g * 16, 16)]
            zi = zv * 2
            cv = plsc.load_gather(tab_v, [zi])
            sv = plsc.load_gather(tab_v, [zi + 1])
            d = g * 32 + iota2
            plsc.store_scatter(out_v, [d], cv)
            plsc.store_scatter(out_v, [d + 1], sv)

        pltpu.sync_copy(out_v, out_hbm.at[pl.ds(off * 2, 2 * _CH)])


def kernel(z, constellation):
    zf = z.reshape(_N)
    tab = constellation.reshape(2 * 16)
    out_flat = pl.kernel(
        _psk_body,
        out_type=jax.ShapeDtypeStruct((2 * _N,), jnp.float32),
        mesh=plsc.VectorSubcoreMesh(
            core_axis_name="c", subcore_axis_name="s",
            num_cores=_NC, num_subcores=_NS,
        ),
        scratch_types=[
            pltpu.VMEM((2 * 16,), jnp.float32),
            pltpu.VMEM((_CH,), jnp.int32),
            pltpu.VMEM((2 * _CH,), jnp.float32),
        ],
        compiler_params=pltpu.CompilerParams(needs_layout_passes=False),
    )(zf, tab)
    return out_flat.reshape(_B, _T, 2)


# speedup vs baseline: 112.3768x; 20.5700x over previous
import jax
import jax.numpy as jnp
from jax import lax
from jax.experimental import pallas as pl
from jax.experimental.pallas import tpu as pltpu
from jax.experimental.pallas import tpu_sc as plsc

_NC, _NS = 2, 16
_NW = _NC * _NS
_B, _T = 16384, 200
_BW = _B // _NW
_RT = 8
_NST = _T // _RT


def _psk_body(zt_hbm, ct_hbm, out_hbm, tabc_v, tabs_v, z_v, o_v):
    wid = lax.axis_index("s") * _NC + lax.axis_index("c")
    b0 = wid * _BW
    pltpu.sync_copy(ct_hbm.at[0], tabc_v)
    pltpu.sync_copy(ct_hbm.at[1], tabs_v)

    @pl.loop(0, _NST)
    def _step(si):
        t0 = si * _RT
        pltpu.sync_copy(zt_hbm.at[pl.ds(t0, _RT), pl.ds(b0, _BW)], z_v)

        @pl.loop(0, _RT)
        def _row(t2):
            @pl.loop(0, _BW // 16)
            def _grp(g):
                zv = z_v[t2, pl.ds(g * 16, 16)]
                cv = plsc.load_gather(tabc_v, [zv])
                sv = plsc.load_gather(tabs_v, [zv])
                bt = g // 8
                j = g % 8
                o_v[t2, 2 * bt, pl.ds(j * 16, 16)] = cv
                o_v[t2, 2 * bt + 1, pl.ds(j * 16, 16)] = sv

        pltpu.sync_copy(o_v, out_hbm.at[pl.ds(t0, _RT), pl.ds(8 * wid, 8), :])


def kernel(z, constellation):
    zt = z.T
    ct = constellation.T
    out3 = pl.kernel(
        _psk_body,
        out_type=jax.ShapeDtypeStruct((_T, 2 * _B // 128, 128), jnp.float32),
        mesh=plsc.VectorSubcoreMesh(
            core_axis_name="c", subcore_axis_name="s",
            num_cores=_NC, num_subcores=_NS,
        ),
        scratch_types=[
            pltpu.VMEM((16,), jnp.float32),
            pltpu.VMEM((16,), jnp.float32),
            pltpu.VMEM((_RT, _BW), jnp.int32),
            pltpu.VMEM((_RT, 8, 128), jnp.float32),
        ],
        compiler_params=pltpu.CompilerParams(
            needs_layout_passes=False, use_tc_tiling_on_sc=True,
        ),
    )(zt, ct)
    out = out3.reshape(_T, 128, 2, 128).transpose(1, 3, 0, 2).reshape(_B, _T, 2)
    return out


# speedup vs baseline: 264.1004x; 2.3501x over previous
import jax
import jax.numpy as jnp
from jax import lax
from jax.experimental import pallas as pl
from jax.experimental.pallas import tpu as pltpu
from jax.experimental.pallas import tpu_sc as plsc

_NC, _NS = 2, 16
_NW = _NC * _NS
_B, _T = 16384, 200
_BW = _B // _NW
_RT = 8
_NST = _T // _RT
_G = _RT * (_BW // 16)


def _psk_body(zt_hbm, ct_hbm, out_hbm,
              tabc_v, tabs_v, z0, z1, o0, o1, si0, si1, so0, so1):
    wid = lax.axis_index("s") * _NC + lax.axis_index("c")
    b0 = wid * _BW
    pltpu.sync_copy(ct_hbm.at[0], tabc_v)
    pltpu.sync_copy(ct_hbm.at[1], tabs_v)

    zbuf, obuf = (z0, z1), (o0, o1)
    zsem, osem = (si0, si1), (so0, so1)

    def in_copy(si, p):
        return pltpu.make_async_copy(
            zt_hbm.at[pl.ds(si * _RT, _RT), pl.ds(b0, _BW)], zbuf[p], zsem[p])

    def out_copy(si, p):
        return pltpu.make_async_copy(
            obuf[p], out_hbm.at[pl.ds(si * _RT, _RT), pl.ds(8 * wid, 8), :],
            osem[p])

    def compute(p):
        zv_ref, ov_ref = zbuf[p], obuf[p]

        @plsc.parallel_loop(0, _G, unroll=8)
        def _grp(i):
            t2 = i >> 5
            g = i & 31
            zv = zv_ref[t2, pl.ds(g * 16, 16)]
            cv = plsc.load_gather(tabc_v, [zv])
            sv = plsc.load_gather(tabs_v, [zv])
            bt = g >> 3
            j = g & 7
            ov_ref[t2, 2 * bt, pl.ds(j * 16, 16)] = cv
            ov_ref[t2, 2 * bt + 1, pl.ds(j * 16, 16)] = sv

    in_copy(0, 0).start()
    in_copy(0, 0).wait()
    in_copy(1, 1).start()
    compute(0)
    out_copy(0, 0).start()
    in_copy(1, 1).wait()
    in_copy(2, 0).start()
    compute(1)
    out_copy(1, 1).start()

    @pl.loop(1, 12)
    def _pair(k):
        si = 2 * k
        in_copy(si, 0).wait()
        in_copy(si + 1, 1).start()
        out_copy(si - 2, 0).wait()
        compute(0)
        out_copy(si, 0).start()
        in_copy(si + 1, 1).wait()
        in_copy(si + 2, 0).start()
        out_copy(si - 1, 1).wait()
        compute(1)
        out_copy(si + 1, 1).start()

    in_copy(24, 0).wait()
    out_copy(22, 0).wait()
    compute(0)
    out_copy(24, 0).start()
    out_copy(23, 1).wait()
    out_copy(24, 0).wait()


def kernel(z, constellation):
    zt = z.T
    ct = constellation.T
    out3 = pl.kernel(
        _psk_body,
        out_type=jax.ShapeDtypeStruct((_T, 2 * _B // 128, 128), jnp.float32),
        mesh=plsc.VectorSubcoreMesh(
            core_axis_name="c", subcore_axis_name="s",
            num_cores=_NC, num_subcores=_NS,
        ),
        scratch_types=[
            pltpu.VMEM((16,), jnp.float32),
            pltpu.VMEM((16,), jnp.float32),
            pltpu.VMEM((_RT, _BW), jnp.int32),
            pltpu.VMEM((_RT, _BW), jnp.int32),
            pltpu.VMEM((_RT, 8, 128), jnp.float32),
            pltpu.VMEM((_RT, 8, 128), jnp.float32),
            pltpu.SemaphoreType.DMA,
            pltpu.SemaphoreType.DMA,
            pltpu.SemaphoreType.DMA,
            pltpu.SemaphoreType.DMA,
        ],
        compiler_params=pltpu.CompilerParams(
            needs_layout_passes=False, use_tc_tiling_on_sc=True,
        ),
    )(zt, ct)
    out = out3.reshape(_T, 128, 2, 128).transpose(1, 3, 0, 2).reshape(_B, _T, 2)
    return out


# speedup vs baseline: 339.1035x; 1.2840x over previous
import jax
import jax.numpy as jnp
from jax import lax
from jax.experimental import pallas as pl
from jax.experimental.pallas import tpu as pltpu
from jax.experimental.pallas import tpu_sc as plsc

_NC, _NS = 2, 16
_NW = _NC * _NS
_B, _T = 16384, 200
_BW = _B // _NW
_RT = 8
_NST = _T // _RT
_G = _RT * (_BW // 16)
_D = 4


def _psk_body(zt_hbm, ct_hbm, out_hbm, tabc_v, tabs_v,
              z0, z1, z2, z3, o0, o1, o2, o3,
              si0, si1, si2, si3, so0, so1, so2, so3):
    wid = lax.axis_index("s") * _NC + lax.axis_index("c")
    b0 = wid * _BW
    pltpu.sync_copy(ct_hbm.at[0], tabc_v)
    pltpu.sync_copy(ct_hbm.at[1], tabs_v)

    zbuf, obuf = (z0, z1, z2, z3), (o0, o1, o2, o3)
    zsem, osem = (si0, si1, si2, si3), (so0, so1, so2, so3)

    def in_copy(si, p):
        return pltpu.make_async_copy(
            zt_hbm.at[pl.ds(si * _RT, _RT), pl.ds(b0, _BW)], zbuf[p], zsem[p])

    def out_copy(si, p):
        return pltpu.make_async_copy(
            obuf[p], out_hbm.at[pl.ds(si * _RT, _RT), pl.ds(8 * wid, 8), :],
            osem[p])

    def compute(p):
        zv_ref, ov_ref = zbuf[p], obuf[p]

        @plsc.parallel_loop(0, _G, unroll=8)
        def _grp(i):
            t2 = i >> 5
            g = i & 31
            zv = zv_ref[t2, pl.ds(g * 16, 16)]
            cv = plsc.load_gather(tabc_v, [zv])
            sv = plsc.load_gather(tabs_v, [zv])
            bt = g >> 3
            j = g & 7
            ov_ref[t2, 2 * bt, pl.ds(j * 16, 16)] = cv
            ov_ref[t2, 2 * bt + 1, pl.ds(j * 16, 16)] = sv

    for p in range(_D):
        in_copy(p, p).start()
    for si in range(_D):
        p = si % _D
        in_copy(si, p).wait()
        compute(p)
        out_copy(si, p).start()
        in_copy(si + _D, p).start()

    @pl.loop(1, 5)
    def _quad(k):
        sb = 4 * k
        for p in range(_D):
            si = sb + p
            in_copy(si, p).wait()
            out_copy(si - _D, p).wait()
            compute(p)
            out_copy(si, p).start()
            in_copy(si + _D, p).start()

    for si in range(20, 25):
        p = si % _D
        in_copy(si, p).wait()
        out_copy(si - _D, p).wait()
        compute(p)
        out_copy(si, p).start()
        if si + _D <= _NST - 1:
            in_copy(si + _D, p).start()
    for si in range(21, 25):
        out_copy(si, si % _D).wait()


def kernel(z, constellation):
    zt = z.T
    ct = constellation.T
    out3 = pl.kernel(
        _psk_body,
        out_type=jax.ShapeDtypeStruct((_T, 2 * _B // 128, 128), jnp.float32),
        mesh=plsc.VectorSubcoreMesh(
            core_axis_name="c", subcore_axis_name="s",
            num_cores=_NC, num_subcores=_NS,
        ),
        scratch_types=(
            [pltpu.VMEM((16,), jnp.float32)] * 2
            + [pltpu.VMEM((_RT, _BW), jnp.int32)] * _D
            + [pltpu.VMEM((_RT, 8, 128), jnp.float32)] * _D
            + [pltpu.SemaphoreType.DMA] * (2 * _D)
        ),
        compiler_params=pltpu.CompilerParams(
            needs_layout_passes=False, use_tc_tiling_on_sc=True,
        ),
    )(zt, ct)
    out = out3.reshape(_T, 128, 2, 128).transpose(1, 3, 0, 2).reshape(_B, _T, 2)
    return out
